# Initial kernel scaffold; baseline (speedup 1.0000x reference)
import jax
import jax.numpy as jnp
from jax.experimental import pallas as pl


def kernel(nodes, t, edges, nbrs, nbr_mask, params):
    raise NotImplementedError("write your pallas kernel here")


# speedup vs baseline: 2.3562x; 2.3562x over previous
import jax
import jax.numpy as jnp
from jax.experimental import pallas as pl
from jax.experimental.pallas import tpu as pltpu
from functools import partial

F32 = jnp.float32
BF16 = jnp.bfloat16


def _silu(x):
    return x * jax.nn.sigmoid(x)


def _ln(x, d):
    c = x - jnp.mean(x, axis=-1, keepdims=True)
    std = jnp.sqrt(jnp.sum(c * c, axis=-1, keepdims=True) / (d - 1))
    return c / jnp.where(std == 0.0, 1.0, std)


def _prep_body(nodes_ref, t_ref,
               ag1, ag2, ag3, agb3, aa1, aa2, aa3,
               fg1, fg2, fg3, fgb3, fa1, fa2, fa3,
               wq, bq,
               nodes_i_ref, q_ref, cond_ref):
    Z, N, D = nodes_ref.shape
    t4 = t_ref[...]
    t16 = t4.astype(BF16)

    def mlp3(x16, w1, w2, w3, b3):
        h = _silu(jnp.dot(x16, w1[...], preferred_element_type=F32))
        h = _silu(jnp.dot(h.astype(BF16), w2[...], preferred_element_type=F32))
        return jnp.dot(h.astype(BF16), w3[...], preferred_element_type=F32) + b3

    gb1 = mlp3(t16, ag1, ag2, ag3, agb3[...])
    a1 = mlp3(t16, aa1, aa2, aa3, 0.0)
    gb2 = mlp3(t16, fg1, fg2, fg3, fgb3[...])
    a2 = mlp3(t16, fa1, fa2, fa3, 0.0)
    alpha1, gamma1 = gb1[:, :D], gb1[:, D:]
    beta1 = a1
    alpha2, gamma2 = gb2[:, :D], gb2[:, D:]
    beta2 = a2
    cond_ref[...] = jnp.concatenate(
        [alpha1, alpha2, gamma2, beta2], axis=-1)

    x = nodes_ref[...]
    xln = _ln(x, D)
    xi = gamma1[:, None, :] * xln + beta1[:, None, :]
    nodes_i_ref[...] = xi.astype(BF16)
    q2 = jnp.dot(xi.reshape(Z * N, D).astype(BF16), wq[...],
                 preferred_element_type=F32) + bq[...]
    q_ref[...] = q2.reshape(Z, N, -1)


def _main_body(edges_ref, nbrs_ref, nodes_i_ref, q_ref, nodes_ref, cond_ref,
               k1, k2, k3, kb3, v1, v2, v3, vb3,
               wk, bk, wv, bv, wo,
               wf1, bf1, wf2, bf2,
               out_ref, *, TN, K, H, DK):
    D = nodes_ref.shape[-1]
    N = nodes_i_ref.shape[1]
    RT = TN * K

    E = edges_ref[0]

    def mlp3(x16, w1, w2, w3, b3):
        h = _silu(jnp.dot(x16, w1[...], preferred_element_type=F32))
        h = _silu(jnp.dot(h.astype(BF16), w2[...], preferred_element_type=F32))
        return jnp.dot(h.astype(BF16), w3[...], preferred_element_type=F32) + b3[...]

    gbK = mlp3(E, k1, k2, k3, kb3)
    gbV = mlp3(E, v1, v2, v3, vb3)

    idx = nbrs_ref[0]
    iota = jax.lax.broadcasted_iota(jnp.int32, (RT, N), 1)
    oh = (iota == idx).astype(BF16)
    nj = jnp.dot(oh, nodes_i_ref[0], preferred_element_type=F32)

    k = gbK[:, :D] * nj + gbK[:, D:]
    v = gbV[:, :D] * nj + gbV[:, D:]
    Kh = jnp.dot(k.astype(BF16), wk[...], preferred_element_type=F32) + bk[...]
    Vh = jnp.dot(v.astype(BF16), wv[...], preferred_element_type=F32) + bv[...]

    qb = q_ref[0]
    scale = DK ** -0.5
    outs = []
    for h in range(H):
        qh = qb[:, h * DK:(h + 1) * DK]
        khh = Kh[:, h * DK:(h + 1) * DK].reshape(TN, K, DK)
        s = jnp.sum(khh * qh[:, None, :], axis=-1) * scale
        m = jnp.max(s, axis=-1, keepdims=True)
        e = jnp.exp(s - m)
        p = e / jnp.sum(e, axis=-1, keepdims=True)
        vhh = Vh[:, h * DK:(h + 1) * DK].reshape(TN, K, DK)
        outs.append(jnp.sum(p[:, :, None] * vhh, axis=1))
    att = jnp.concatenate(outs, axis=-1)
    att = jnp.dot(att.astype(BF16), wo[...], preferred_element_type=F32)

    cz = cond_ref[0]
    alpha1 = cz[:, 0 * D:1 * D]
    alpha2 = cz[:, 1 * D:2 * D]
    gamma2 = cz[:, 2 * D:3 * D]
    beta2 = cz[:, 3 * D:4 * D]

    x = nodes_ref[0] + alpha1 * att
    x2 = gamma2 * _ln(x, D) + beta2
    hh = _silu(jnp.dot(x2.astype(BF16), wf1[...], preferred_element_type=F32)
               + bf1[...])
    f = jnp.dot(hh.astype(BF16), wf2[...], preferred_element_type=F32) + bf2[...]
    out_ref[0] = x + alpha2 * f


def _b16(x):
    return x.astype(BF16)


def _row(x):
    return x.reshape(1, -1).astype(F32)


def kernel(nodes, t, edges, nbrs, nbr_mask, params):
    Z, N, D = nodes.shape
    K = nbrs.shape[2]
    H, _, DK = params[0]['q_proj'].shape
    TN = 64
    NT = N // TN
    RT = TN * K
    RS = RT // 256

    edges_r = edges.reshape(Z, N * K, D).astype(BF16)
    nbrs_r = nbrs.reshape(Z, N * K, 1)
    t2 = t.reshape(Z, D)

    full = lambda a: pl.BlockSpec(a.shape, lambda *_: (0,) * a.ndim)

    x = nodes
    for p in params:
        pw = []
        for nm in ('an_gb', 'an_a', 'fn_gb', 'fn_a'):
            mats = [_b16(w) for w, b in p[nm]]
            pw.extend(mats)
            if nm.endswith('gb'):
                pw.append(_row(p[nm][-1][1]))
        wq = _b16(jnp.transpose(p['q_proj'], (1, 0, 2)).reshape(D, H * DK))
        bq = _row(p['q_bias'])
        prep_in = [x, t2] + pw + [wq, bq]
        nodes_i, q, cond = pl.pallas_call(
            _prep_body,
            grid=(1,),
            in_specs=[full(a) for a in prep_in],
            out_specs=[
                pl.BlockSpec((Z, N, D), lambda i: (0, 0, 0)),
                pl.BlockSpec((Z, N, H * DK), lambda i: (0, 0, 0)),
                pl.BlockSpec((Z, 4 * D), lambda i: (0, 0)),
            ],
            out_shape=[
                jax.ShapeDtypeStruct((Z, N, D), BF16),
                jax.ShapeDtypeStruct((Z, N, H * DK), F32),
                jax.ShapeDtypeStruct((Z, 4 * D), F32),
            ],
        )(*prep_in)
        cond3 = cond.reshape(Z, 1, 4 * D)

        kmats = [_b16(w) for w, _ in p['fK']] + [_row(p['fK'][-1][1])]
        vmats = [_b16(w) for w, _ in p['fV']] + [_row(p['fV'][-1][1])]
        wk = _b16(jnp.transpose(p['k_proj'], (1, 0, 2)).reshape(D, H * DK))
        bk = _row(p['k_bias'])
        wv = _b16(jnp.transpose(p['v_proj'], (1, 0, 2)).reshape(D, H * DK))
        bv = _row(p['v_bias'])
        wo = _b16(p['out_W'].reshape(DK, H, D).transpose(1, 0, 2).reshape(D, D))
        wf1, bf1 = _b16(p['ffn'][0][0]), _row(p['ffn'][0][1])
        wf2, bf2 = _b16(p['ffn'][1][0]), _row(p['ffn'][1][1])
        wlist = kmats + vmats + [wk, bk, wv, bv, wo, wf1, bf1, wf2, bf2]

        x = pl.pallas_call(
            partial(_main_body, TN=TN, K=K, H=H, DK=DK),
            grid=(Z, NT),
            in_specs=[
                pl.BlockSpec((1, RT, D), lambda z, c: (z, c, 0)),
                pl.BlockSpec((1, RT, 1), lambda z, c: (z, c, 0)),
                pl.BlockSpec((1, N, D), lambda z, c: (z, 0, 0)),
                pl.BlockSpec((1, TN, H * DK), lambda z, c: (z, c, 0)),
                pl.BlockSpec((1, TN, D), lambda z, c: (z, c, 0)),
                pl.BlockSpec((1, 1, 4 * D), lambda z, c: (z, 0, 0)),
            ] + [pl.BlockSpec(a.shape, lambda z, c, nd=a.ndim: (0,) * nd)
                 for a in wlist],
            out_specs=pl.BlockSpec((1, TN, D), lambda z, c: (z, c, 0)),
            out_shape=jax.ShapeDtypeStruct((Z, N, D), F32),
        )(edges_r, nbrs_r, nodes_i, q, x, cond3, *wlist)
    return x


# speedup vs baseline: 5.8375x; 2.4775x over previous
import jax
import jax.numpy as jnp
from jax import lax
from jax.experimental import pallas as pl
from jax.experimental.pallas import tpu as pltpu
from jax.experimental.pallas import tpu_sc as plsc
from functools import partial

F32 = jnp.float32
BF16 = jnp.bfloat16


def _silu(x):
    return x * jax.nn.sigmoid(x)


def _silu16(x):
    return x / (jnp.exp(-x) + jnp.bfloat16(1.0))


def _ln(x, d):
    c = x - jnp.mean(x, axis=-1, keepdims=True)
    std = jnp.sqrt(jnp.sum(c * c, axis=-1, keepdims=True) / (d - 1))
    return c / jnp.where(std == 0.0, 1.0, std)


def _prep_body(nodes_ref, t_ref,
               ag1, ag2, ag3, agb3, aa1, aa2, aa3,
               fg1, fg2, fg3, fgb3, fa1, fa2, fa3,
               wq, bq,
               nodes_i_ref, q_ref, cond_ref):
    Z, N, D = nodes_ref.shape
    t4 = t_ref[...]
    t16 = t4.astype(BF16)

    def mlp3(x16, w1, w2, w3, b3):
        h = _silu(jnp.dot(x16, w1[...], preferred_element_type=F32))
        h = _silu(jnp.dot(h.astype(BF16), w2[...], preferred_element_type=F32))
        return jnp.dot(h.astype(BF16), w3[...], preferred_element_type=F32) + b3

    gb1 = mlp3(t16, ag1, ag2, ag3, agb3[...])
    a1 = mlp3(t16, aa1, aa2, aa3, 0.0)
    gb2 = mlp3(t16, fg1, fg2, fg3, fgb3[...])
    a2 = mlp3(t16, fa1, fa2, fa3, 0.0)
    alpha1, gamma1 = gb1[:, :D], gb1[:, D:]
    beta1 = a1
    alpha2, gamma2 = gb2[:, :D], gb2[:, D:]
    beta2 = a2
    cond_ref[...] = jnp.concatenate(
        [alpha1, alpha2, gamma2, beta2], axis=-1)

    x = nodes_ref[...]
    xln = _ln(x, D)
    xi = gamma1[:, None, :] * xln + beta1[:, None, :]
    ai = jax.lax.bitcast_convert_type(xi[..., :D // 2], jnp.int32)
    bi = jax.lax.bitcast_convert_type(xi[..., D // 2:], jnp.int32)
    lo = jax.lax.shift_right_logical(ai + jnp.int32(0x8000), 16)
    hi = jnp.bitwise_and(bi + jnp.int32(0x8000), jnp.int32(-65536))
    nodes_i_ref[...] = jnp.bitwise_or(hi, lo)
    q2 = jnp.dot(xi.reshape(Z * N, D).astype(BF16), wq[...],
                 preferred_element_type=F32) + bq[...]
    q_ref[...] = q2.reshape(Z, N, -1)


def _idx_body(nbrs_ref, out_ref, *, N):
    Z, NK = nbrs_ref.shape
    zix = jax.lax.broadcasted_iota(jnp.int32, (Z, NK), 0)
    out_ref[...] = nbrs_ref[...] + zix * N


def _sc_gather(table, idx, CH=256):
    B = idx.shape[0]
    tail = table.shape[1:]
    NW = 32
    b_per_w = B // NW
    nchunks = b_per_w // CH
    mesh = plsc.VectorSubcoreMesh(core_axis_name="c", subcore_axis_name="s")

    @partial(
        pl.kernel, mesh=mesh,
        out_type=jax.ShapeDtypeStruct((B,) + tail, table.dtype),
        scratch_types=[
            pltpu.VMEM((CH,), jnp.int32),
            pltpu.VMEM((CH,), jnp.int32),
            pltpu.VMEM((CH,) + tail, table.dtype),
            pltpu.VMEM((CH,) + tail, table.dtype),
            pltpu.SemaphoreType.DMA,
            pltpu.SemaphoreType.DMA,
            pltpu.SemaphoreType.DMA,
            pltpu.SemaphoreType.DMA,
        ],
    )
    def gk(table_hbm, idx_hbm, out_hbm, idxv0, idxv1, rows0, rows1,
           g0, g1, w0, w1):
        wid = lax.axis_index("s") * 2 + lax.axis_index("c")
        base = wid * b_per_w
        idxv = (idxv0, idxv1)
        rows = (rows0, rows1)
        gsem = (g0, g1)
        wsem = (w0, w1)
        gcp = [None, None]
        wcp = [None, None]
        for i in range(nchunks):
            b = i & 1
            if i >= 2:
                wcp[b].wait()
            off = base + i * CH
            pltpu.sync_copy(idx_hbm.at[pl.ds(off, CH)], idxv[b])
            gcp[b] = pltpu.async_copy(table_hbm.at[idxv[b]], rows[b], gsem[b])
            if i >= 1:
                pb = 1 - b
                gcp[pb].wait()
                wcp[pb] = pltpu.async_copy(
                    rows[pb], out_hbm.at[pl.ds(base + (i - 1) * CH, CH)],
                    wsem[pb])
        lb = (nchunks - 1) & 1
        gcp[lb].wait()
        wcp[lb] = pltpu.async_copy(
            rows[lb], out_hbm.at[pl.ds(base + (nchunks - 1) * CH, CH)],
            wsem[lb])
        if nchunks > 1:
            wcp[1 - lb].wait()
        wcp[lb].wait()

    return gk(table, idx)


def _main_body(edges_ref, nodes_j_ref, q_ref, nodes_ref, cond_ref,
               k1, k2, k3, kb3, v1, v2, v3, vb3,
               wk, bk, wv, bv, wo,
               wf1, bf1, wf2, bf2,
               rep, rept, g64,
               out_ref, *, TN, K, H, DK):
    D = nodes_ref.shape[-1]
    RT = TN * K

    E = edges_ref[0]

    def mlp3(x16, w1, w2, w3, b3):
        h = _silu16(jnp.dot(x16, w1[...], preferred_element_type=F32).astype(BF16))
        h = _silu16(jnp.dot(h, w2[...], preferred_element_type=F32).astype(BF16))
        return jnp.dot(h, w3[...], preferred_element_type=F32) + b3[...]

    gbK = mlp3(E, k1, k2, k3, kb3)
    gbV = mlp3(E, v1, v2, v3, vb3)

    w = nodes_j_ref[0]
    nja = jax.lax.bitcast_convert_type(jax.lax.shift_left(w, 16), F32)
    njb = jax.lax.bitcast_convert_type(
        jnp.bitwise_and(w, jnp.int32(-65536)), F32)
    nj = jnp.concatenate([nja, njb], axis=-1)

    k = gbK[:, :D] * nj + gbK[:, D:]
    v = gbV[:, :D] * nj + gbV[:, D:]
    Kh = jnp.dot(k.astype(BF16), wk[...], preferred_element_type=F32) + bk[...]
    Vh = jnp.dot(v.astype(BF16), wv[...], preferred_element_type=F32) + bv[...]

    qs = (q_ref[0] * (DK ** -0.5)).astype(BF16)
    qrep = jnp.dot(rep[...], qs, preferred_element_type=F32)
    prod = (Kh * qrep).astype(BF16)
    s2 = jnp.dot(prod, g64[...], preferred_element_type=F32)
    m2 = jnp.max(s2, axis=0, keepdims=True)
    e2 = jnp.exp(s2 - m2)
    ev = (e2 * Vh).astype(BF16)
    num = jnp.dot(rept[...], ev, preferred_element_type=F32)
    den = jnp.dot(rept[...], e2.astype(BF16), preferred_element_type=F32)
    att = num / jnp.maximum(den, 1e-30)
    att = jnp.dot(att.astype(BF16), wo[...], preferred_element_type=F32)

    cz = cond_ref[0]
    alpha1 = cz[:, 0 * D:1 * D]
    alpha2 = cz[:, 1 * D:2 * D]
    gamma2 = cz[:, 2 * D:3 * D]
    beta2 = cz[:, 3 * D:4 * D]

    x = nodes_ref[0] + alpha1 * att
    x2 = gamma2 * _ln(x, D) + beta2
    hh = _silu(jnp.dot(x2.astype(BF16), wf1[...], preferred_element_type=F32)
               + bf1[...])
    f = jnp.dot(hh.astype(BF16), wf2[...], preferred_element_type=F32) + bf2[...]
    out_ref[0] = x + alpha2 * f


def _b16(x):
    return x.astype(BF16)


def _row(x):
    return x.reshape(1, -1).astype(F32)


def kernel(nodes, t, edges, nbrs, nbr_mask, params):
    Z, N, D = nodes.shape
    K = nbrs.shape[2]
    H, _, DK = params[0]['q_proj'].shape
    TN = 128
    NT = N // TN
    RT = TN * K
    RS = RT // 256

    edges_r = edges.reshape(Z, N * K, D).astype(BF16)
    t2 = t.reshape(Z, D)

    full = lambda a: pl.BlockSpec(a.shape, lambda *_: (0,) * a.ndim)

    nbrs2 = nbrs.reshape(Z, N * K)
    idx_g = pl.pallas_call(
        partial(_idx_body, N=N),
        grid=(1,),
        in_specs=[full(nbrs2)],
        out_specs=pl.BlockSpec((Z, N * K), lambda i: (0, 0)),
        out_shape=jax.ShapeDtypeStruct((Z, N * K), jnp.int32),
    )(nbrs2).reshape(Z * N * K)

    x = nodes
    for p in params:
        pw = []
        for nm in ('an_gb', 'an_a', 'fn_gb', 'fn_a'):
            mats = [_b16(w) for w, b in p[nm]]
            pw.extend(mats)
            if nm.endswith('gb'):
                pw.append(_row(p[nm][-1][1]))
        wq = _b16(jnp.transpose(p['q_proj'], (1, 0, 2)).reshape(D, H * DK))
        bq = _row(p['q_bias'])
        prep_in = [x, t2] + pw + [wq, bq]
        nodes_i, q, cond = pl.pallas_call(
            _prep_body,
            grid=(1,),
            in_specs=[full(a) for a in prep_in],
            out_specs=[
                pl.BlockSpec((Z, N, D // 2), lambda i: (0, 0, 0)),
                pl.BlockSpec((Z, N, H * DK), lambda i: (0, 0, 0)),
                pl.BlockSpec((Z, 4 * D), lambda i: (0, 0)),
            ],
            out_shape=[
                jax.ShapeDtypeStruct((Z, N, D // 2), jnp.int32),
                jax.ShapeDtypeStruct((Z, N, H * DK), F32),
                jax.ShapeDtypeStruct((Z, 4 * D), F32),
            ],
        )(*prep_in)
        cond3 = cond.reshape(Z, 1, 4 * D)

        kmats = [_b16(w) for w, _ in p['fK']] + [_row(p['fK'][-1][1])]
        vmats = [_b16(w) for w, _ in p['fV']] + [_row(p['fV'][-1][1])]
        wk = _b16(jnp.transpose(p['k_proj'], (1, 0, 2)).reshape(D, H * DK))
        bk = _row(p['k_bias'])
        wv = _b16(jnp.transpose(p['v_proj'], (1, 0, 2)).reshape(D, H * DK))
        bv = _row(p['v_bias'])
        wo = _b16(p['out_W'].reshape(DK, H, D).transpose(1, 0, 2).reshape(D, D))
        wf1, bf1 = _b16(p['ffn'][0][0]), _row(p['ffn'][0][1])
        wf2, bf2 = _b16(p['ffn'][1][0]), _row(p['ffn'][1][1])
        rep = (jnp.arange(RT)[:, None] // K
               == jnp.arange(TN)[None, :]).astype(BF16)
        rept = rep.T
        g64 = (jnp.arange(D)[:, None] // DK
               == jnp.arange(H * DK)[None, :] // DK).astype(BF16)
        wlist = kmats + vmats + [wk, bk, wv, bv, wo, wf1, bf1, wf2, bf2,
                                 rep, rept, g64]

        table = nodes_i.reshape(Z * N, D // 2)
        ZH = Z // 2
        idx2 = idx_g.reshape(2, ZH * N * K)
        halves = []
        for h in range(2):
            nodes_j = _sc_gather(table, idx2[h])
            nodes_j = nodes_j.reshape(ZH, N * K, D // 2)
            xh = pl.pallas_call(
                partial(_main_body, TN=TN, K=K, H=H, DK=DK),
                grid=(ZH, NT),
                in_specs=[
                    pl.BlockSpec((1, RT, D),
                                 lambda z, c, h=h: (z + h * ZH, c, 0)),
                    pl.BlockSpec((1, RT, D // 2), lambda z, c: (z, c, 0)),
                    pl.BlockSpec((1, TN, H * DK),
                                 lambda z, c, h=h: (z + h * ZH, c, 0)),
                    pl.BlockSpec((1, TN, D),
                                 lambda z, c, h=h: (z + h * ZH, c, 0)),
                    pl.BlockSpec((1, 1, 4 * D),
                                 lambda z, c, h=h: (z + h * ZH, 0, 0)),
                ] + [pl.BlockSpec(a.shape, lambda z, c, nd=a.ndim: (0,) * nd)
                     for a in wlist],
                out_specs=pl.BlockSpec((1, TN, D), lambda z, c: (z, c, 0)),
                out_shape=jax.ShapeDtypeStruct((ZH, N, D), F32),
            )(edges_r, nodes_j, q, x, cond3, *wlist)
            halves.append(xh)
        x = jnp.concatenate(halves, axis=0)
    return x
